# Initial kernel scaffold; baseline (speedup 1.0000x reference)
import jax
import jax.numpy as jnp
from jax.experimental import pallas as pl


def kernel(x, edge_index, W1s, W1d, a1s, a1d, b1, W2s, W2d, a2s, a2d, b2):
    raise NotImplementedError("write your pallas kernel here")


# speedup vs baseline: 12.3641x; 12.3641x over previous
import functools

import jax
import jax.numpy as jnp
from jax import lax
from jax.experimental import pallas as pl
from jax.experimental.pallas import tpu as pltpu
from jax.experimental.pallas import tpu_sc as plsc

N = 10000
NPAD = 10240
E = 160000
D = 256
F = 256
FH = 128
NEG_SLOPE = 0.2

NC = 2
NS = 16
RPT = NPAD // NS

EPW = E // (NC * NS)
VPW = 313
EPAD = VPW * 16

EPT = E // NS
C = 320
NCHUNK = EPT // C
CT = EPT - NCHUNK * C
SUB = 80
FIN = 80


def _tc_first(x_ref, ws_ref, wd_ref, as_ref, ad_ref, xs_ref, asrc_ref, adst_ref):
    i = pl.program_id(0)
    xb = x_ref[...]
    xs = jnp.dot(xb, ws_ref[...], preferred_element_type=jnp.float32)
    xs_ref[...] = xs
    asrc_ref[pl.ds(i, 1), :] = jnp.sum(xs * as_ref[...], axis=1).reshape(1, 1024)
    wdv = jnp.sum(wd_ref[...] * ad_ref[...], axis=1)
    adst_ref[pl.ds(i, 1), :] = jnp.sum(xb * wdv[None, :], axis=1).reshape(1, 1024)


def _tc_second(h_ref, ws_ref, wd_ref, as_ref, ad_ref, xs_ref, asrc_ref, adst_ref):
    lo = jax.nn.relu(h_ref[0])
    hi = jax.nn.relu(h_ref[1])
    w = ws_ref[...]
    xs = (jnp.dot(lo, w[:FH, :], preferred_element_type=jnp.float32)
          + jnp.dot(hi, w[FH:, :], preferred_element_type=jnp.float32))
    i = pl.program_id(0)
    xs_ref[...] = xs
    asrc_ref[pl.ds(i, 1), :] = jnp.sum(xs * as_ref[...], axis=1).reshape(1, 1024)
    wdv = jnp.sum(wd_ref[...] * ad_ref[...], axis=1)
    adst = (jnp.sum(lo * wdv[None, :FH], axis=1)
            + jnp.sum(hi * wdv[None, FH:], axis=1))
    adst_ref[pl.ds(i, 1), :] = adst.reshape(1, 1024)


def _sc_soft(asrc, adst, srch, dsth, ex_out, dparts,
             dens, asrc_v, adst_v, den_local, dbuf, src_v, dst_v, ex_v, d640):
    c = lax.axis_index("c")
    s = lax.axis_index("s")
    wid = c * NS + s
    off = wid * EPW
    r0 = s * RPT
    z16 = jnp.zeros((16,), jnp.float32)
    iota = lax.iota(jnp.int32, 16)

    pltpu.sync_copy(asrc, asrc_v)
    pltpu.sync_copy(adst, adst_v)
    pltpu.sync_copy(srch.at[pl.ds(off, EPW)], src_v.at[pl.ds(0, EPW)])
    pltpu.sync_copy(dsth.at[pl.ds(off, EPW)], dst_v.at[pl.ds(0, EPW)])

    def _z_den(i, _):
        den_local[pl.ds(i * 16, 16)] = z16
        return 0
    lax.fori_loop(0, NPAD // 16, _z_den, 0)

    tl = EPAD - 16
    tv = src_v[pl.ds(tl, 16)]
    src_v[pl.ds(tl, 16)] = jnp.where(iota < EPW - tl, tv, 0)
    tv = dst_v[pl.ds(tl, 16)]
    dst_v[pl.ds(tl, 16)] = jnp.where(iota < EPW - tl, tv, 0)

    def _edge(i, _):
        o = i * 16
        s16 = src_v[pl.ds(o, 16)]
        d16 = dst_v[pl.ds(o, 16)]
        av = plsc.load_gather(asrc_v, [s16])
        dv = plsc.load_gather(adst_v, [d16])
        e = av + dv
        e = jnp.where(e > 0, e, e * NEG_SLOPE)
        ex = jnp.exp(e)
        ex = jnp.where(o + iota < EPW, ex, 0.0)
        ex_v[pl.ds(o, 16)] = ex
        plsc.addupdate_scatter(den_local, [d16], ex)
        return 0
    lax.fori_loop(0, VPW, _edge, 0)

    pltpu.sync_copy(ex_v.at[pl.ds(0, EPW)], ex_out.at[pl.ds(off, EPW)])
    pltpu.sync_copy(den_local, dens.at[s])
    plsc.subcore_barrier()

    for r in range(NS):
        pltpu.sync_copy(dens.at[r, pl.ds(r0, RPT)], dbuf.at[r])

    def _red(i, _):
        acc = dbuf[0, pl.ds(i * 16, 16)]
        for r in range(1, NS):
            acc = acc + dbuf[r, pl.ds(i * 16, 16)]
        d640[pl.ds(i * 16, 16)] = acc
        return 0
    lax.fori_loop(0, RPT // 16, _red, 0)
    pltpu.sync_copy(d640, dparts.at[c, pl.ds(r0, RPT)])


_sc_soft_call = functools.partial(
    pl.kernel,
    out_type=(jax.ShapeDtypeStruct((E,), jnp.float32),
              jax.ShapeDtypeStruct((NC, NPAD), jnp.float32)),
    mesh=plsc.VectorSubcoreMesh(
        core_axis_name="c", subcore_axis_name="s",
        num_cores=NC, num_subcores=NS),
    compiler_params=pltpu.CompilerParams(needs_layout_passes=False),
    scratch_types=[
        pltpu.VMEM_SHARED((NS, NPAD), jnp.float32),
        pltpu.VMEM((NPAD,), jnp.float32),
        pltpu.VMEM((NPAD,), jnp.float32),
        pltpu.VMEM((NPAD,), jnp.float32),
        pltpu.VMEM((NS, RPT), jnp.float32),
        pltpu.VMEM((EPAD,), jnp.int32),
        pltpu.VMEM((EPAD,), jnp.int32),
        pltpu.VMEM((EPAD,), jnp.float32),
        pltpu.VMEM((RPT,), jnp.float32),
    ],
)(_sc_soft)


def _sc_agg(xs2h, srch, dsth, exh, dparts, bias, out,
            ysum, rows, didx, src_v, dst_v, ex_v, dpb0, dpb1, dsum, bias_v, sem):
    c = lax.axis_index("c")
    s = lax.axis_index("s")
    z16 = jnp.zeros((16,), jnp.float32)
    r0 = s * RPT

    pltpu.sync_copy(bias.at[pl.ds(c * FH, FH)], bias_v)

    def _z_rows(j, _):
        for m in range(FH // 16):
            rows[j, pl.ds(m * 16, 16)] = z16
        return 0
    lax.fori_loop(0, C, _z_rows, 0)

    pltpu.sync_copy(rows.at[pl.ds(0, RPT // 2)], ysum.at[pl.ds(r0, RPT // 2)])
    pltpu.sync_copy(rows.at[pl.ds(0, RPT // 2)],
                    ysum.at[pl.ds(r0 + RPT // 2, RPT // 2)])
    plsc.subcore_barrier()

    def _do_chunk(off, size):
        pltpu.sync_copy(srch.at[pl.ds(off, size)], src_v.at[pl.ds(0, size)])
        pltpu.sync_copy(dsth.at[pl.ds(off, size)], dst_v.at[pl.ds(0, size)])
        pltpu.sync_copy(exh.at[pl.ds(off, size)], ex_v.at[pl.ds(0, size)])
        descs = []
        for q in range(size // 16):
            sidx = src_v[pl.ds(q * 16, 16)] * 2 + c
            descs.append(pltpu.async_copy(
                xs2h.at[sidx], rows.at[pl.ds(q * 16, 16)], sem))
        for qa in range(size // SUB):
            for qb in range(SUB // 16):
                didx[qa, pl.ds(qb * 16, 16)] = dst_v[pl.ds(qa * SUB + qb * 16, 16)]
        for dd in descs:
            dd.wait()

        def _scale(j, _):
            al = ex_v[pl.ds(j, 16)][0]
            for m in range(FH // 16):
                rows[j, pl.ds(m * 16, 16)] = rows[j, pl.ds(m * 16, 16)] * al
            return 0
        lax.fori_loop(0, size, _scale, 0)

        for qa in range(size // SUB):
            pltpu.sync_copy(rows.at[pl.ds(qa * SUB, SUB)],
                            ysum.at[didx.at[qa]], add=True)

    def _chunk(k, _):
        _do_chunk(s * EPT + k * C, C)
        return 0
    lax.fori_loop(0, NCHUNK, _chunk, 0)
    _do_chunk(s * EPT + NCHUNK * C, CT)

    plsc.subcore_barrier()

    pltpu.sync_copy(dparts.at[0, pl.ds(r0, RPT)], dpb0)
    pltpu.sync_copy(dparts.at[1, pl.ds(r0, RPT)], dpb1)

    def _rcp(i, _):
        p = dpb0[pl.ds(i * 16, 16)] + dpb1[pl.ds(i * 16, 16)]
        dsum[pl.ds(i * 16, 16)] = 1.0 / (p + 1e-16)
        return 0
    lax.fori_loop(0, RPT // 16, _rcp, 0)

    for f in range(RPT // FIN):
        rr = r0 + f * FIN
        pltpu.sync_copy(ysum.at[pl.ds(rr, FIN)], rows.at[pl.ds(0, FIN)])

        def _frow(j, _):
            sc = dsum[pl.ds(f * FIN + j, 16)][0]
            for m in range(FH // 16):
                rows[j, pl.ds(m * 16, 16)] = (
                    rows[j, pl.ds(m * 16, 16)] * sc + bias_v[pl.ds(m * 16, 16)])
            return 0
        lax.fori_loop(0, FIN, _frow, 0)
        pltpu.sync_copy(rows.at[pl.ds(0, FIN)], out.at[c, pl.ds(rr, FIN)])


_sc_agg_call = functools.partial(
    pl.kernel,
    out_type=jax.ShapeDtypeStruct((NC, NPAD, FH), jnp.float32),
    mesh=plsc.VectorSubcoreMesh(
        core_axis_name="c", subcore_axis_name="s",
        num_cores=NC, num_subcores=NS),
    compiler_params=pltpu.CompilerParams(needs_layout_passes=False),
    scratch_types=[
        pltpu.VMEM_SHARED((NPAD, FH), jnp.float32),
        pltpu.VMEM((C, FH), jnp.float32),
        pltpu.VMEM((C // SUB, SUB), jnp.int32),
        pltpu.VMEM((C,), jnp.int32),
        pltpu.VMEM((C,), jnp.int32),
        pltpu.VMEM((C + 16,), jnp.float32),
        pltpu.VMEM((RPT,), jnp.float32),
        pltpu.VMEM((RPT,), jnp.float32),
        pltpu.VMEM((RPT + 16,), jnp.float32),
        pltpu.VMEM((FH,), jnp.float32),
        pltpu.SemaphoreType.DMA,
    ],
)(_sc_agg)


def _tc_layer1(x_pad, W1s, W1d, a1s, a1d):
    return pl.pallas_call(
        _tc_first,
        grid=(NPAD // 1024,),
        in_specs=[
            pl.BlockSpec((1024, D), lambda i: (i, 0)),
            pl.BlockSpec((D, F), lambda i: (0, 0)),
            pl.BlockSpec((D, F), lambda i: (0, 0)),
            pl.BlockSpec((1, F), lambda i: (0, 0)),
            pl.BlockSpec((1, F), lambda i: (0, 0)),
        ],
        out_specs=[
            pl.BlockSpec((1024, F), lambda i: (i, 0)),
            pl.BlockSpec((NPAD // 1024, 1024), lambda i: (0, 0)),
            pl.BlockSpec((NPAD // 1024, 1024), lambda i: (0, 0)),
        ],
        out_shape=[
            jax.ShapeDtypeStruct((NPAD, F), jnp.float32),
            jax.ShapeDtypeStruct((NPAD // 1024, 1024), jnp.float32),
            jax.ShapeDtypeStruct((NPAD // 1024, 1024), jnp.float32),
        ],
    )(x_pad, W1s, W1d, a1s.reshape(1, F), a1d.reshape(1, F))


def _tc_layer2(h, W2s, W2d, a2s, a2d):
    return pl.pallas_call(
        _tc_second,
        grid=(NPAD // 1024,),
        in_specs=[
            pl.BlockSpec((NC, 1024, FH), lambda i: (0, i, 0)),
            pl.BlockSpec((F, F), lambda i: (0, 0)),
            pl.BlockSpec((F, F), lambda i: (0, 0)),
            pl.BlockSpec((1, F), lambda i: (0, 0)),
            pl.BlockSpec((1, F), lambda i: (0, 0)),
        ],
        out_specs=[
            pl.BlockSpec((1024, F), lambda i: (i, 0)),
            pl.BlockSpec((NPAD // 1024, 1024), lambda i: (0, 0)),
            pl.BlockSpec((NPAD // 1024, 1024), lambda i: (0, 0)),
        ],
        out_shape=[
            jax.ShapeDtypeStruct((NPAD, F), jnp.float32),
            jax.ShapeDtypeStruct((NPAD // 1024, 1024), jnp.float32),
            jax.ShapeDtypeStruct((NPAD // 1024, 1024), jnp.float32),
        ],
    )(h, W2s, W2d, a2s.reshape(1, F), a2d.reshape(1, F))


def kernel(x, edge_index, W1s, W1d, a1s, a1d, b1, W2s, W2d, a2s, a2d, b2):
    x_pad = jnp.pad(x, ((0, NPAD - N), (0, 0)))
    src = edge_index[0]
    dst = edge_index[1]

    xs1, asrc1, adst1 = _tc_layer1(x_pad, W1s, W1d, a1s, a1d)
    ex1, dparts1 = _sc_soft_call(asrc1.reshape(NPAD), adst1.reshape(NPAD),
                                 src, dst)
    h = _sc_agg_call(xs1.reshape(2 * NPAD, FH), src, dst, ex1, dparts1, b1)

    xs2, asrc2, adst2 = _tc_layer2(h, W2s, W2d, a2s, a2d)
    ex2, dparts2 = _sc_soft_call(asrc2.reshape(NPAD), adst2.reshape(NPAD),
                                 src, dst)
    out2 = _sc_agg_call(xs2.reshape(2 * NPAD, FH), src, dst, ex2, dparts2, b2)

    return out2.transpose(1, 0, 2).reshape(NPAD, F)[:N]


# speedup vs baseline: 17.4086x; 1.4080x over previous
import functools

import jax
import jax.numpy as jnp
from jax import lax
from jax.experimental import pallas as pl
from jax.experimental.pallas import tpu as pltpu
from jax.experimental.pallas import tpu_sc as plsc

N = 10000
NPAD = 10240
E = 160000
D = 256
F = 256
FH = 128
NEG_SLOPE = 0.2

NC = 2
NS = 16
RPT = NPAD // NS

EPW = E // (NC * NS)
VPW = 313
EPAD = VPW * 16

EPT = E // NS
C = 320
PC = 160
NPC = EPT // PC
CT = EPT - NPC * PC
SUB = 80
FIN = 80


def _tc_first(x_ref, ws_ref, wd_ref, as_ref, ad_ref, xs_ref, asrc_ref, adst_ref):
    i = pl.program_id(0)
    xb = x_ref[...]
    xs = jnp.dot(xb, ws_ref[...], preferred_element_type=jnp.float32)
    xs_ref[...] = xs
    asrc_ref[pl.ds(i, 1), :] = jnp.sum(xs * as_ref[...], axis=1).reshape(1, 1024)
    wdv = jnp.sum(wd_ref[...] * ad_ref[...], axis=1)
    adst_ref[pl.ds(i, 1), :] = jnp.sum(xb * wdv[None, :], axis=1).reshape(1, 1024)


def _tc_second(h_ref, ws_ref, wd_ref, as_ref, ad_ref, xs_ref, asrc_ref, adst_ref):
    lo = jax.nn.relu(h_ref[0])
    hi = jax.nn.relu(h_ref[1])
    w = ws_ref[...]
    xs = (jnp.dot(lo, w[:FH, :], preferred_element_type=jnp.float32)
          + jnp.dot(hi, w[FH:, :], preferred_element_type=jnp.float32))
    i = pl.program_id(0)
    xs_ref[...] = xs
    asrc_ref[pl.ds(i, 1), :] = jnp.sum(xs * as_ref[...], axis=1).reshape(1, 1024)
    wdv = jnp.sum(wd_ref[...] * ad_ref[...], axis=1)
    adst = (jnp.sum(lo * wdv[None, :FH], axis=1)
            + jnp.sum(hi * wdv[None, FH:], axis=1))
    adst_ref[pl.ds(i, 1), :] = adst.reshape(1, 1024)


def _sc_soft(asrc, adst, srch, dsth, ex_out, dparts,
             dens, asrc_v, adst_v, den_local, dbuf, src_v, dst_v, ex_v, d640):
    c = lax.axis_index("c")
    s = lax.axis_index("s")
    wid = c * NS + s
    off = wid * EPW
    r0 = s * RPT
    z16 = jnp.zeros((16,), jnp.float32)
    iota = lax.iota(jnp.int32, 16)

    pltpu.sync_copy(asrc, asrc_v)
    pltpu.sync_copy(adst, adst_v)
    pltpu.sync_copy(srch.at[pl.ds(off, EPW)], src_v.at[pl.ds(0, EPW)])
    pltpu.sync_copy(dsth.at[pl.ds(off, EPW)], dst_v.at[pl.ds(0, EPW)])

    def _z_den(i, _):
        den_local[pl.ds(i * 16, 16)] = z16
        return 0
    lax.fori_loop(0, NPAD // 16, _z_den, 0)

    tl = EPAD - 16
    tv = src_v[pl.ds(tl, 16)]
    src_v[pl.ds(tl, 16)] = jnp.where(iota < EPW - tl, tv, 0)
    tv = dst_v[pl.ds(tl, 16)]
    dst_v[pl.ds(tl, 16)] = jnp.where(iota < EPW - tl, tv, 0)

    def _edge(i, _):
        o = i * 16
        s16 = src_v[pl.ds(o, 16)]
        d16 = dst_v[pl.ds(o, 16)]
        av = plsc.load_gather(asrc_v, [s16])
        dv = plsc.load_gather(adst_v, [d16])
        e = av + dv
        e = jnp.where(e > 0, e, e * NEG_SLOPE)
        ex = jnp.exp(e)
        ex = jnp.where(o + iota < EPW, ex, 0.0)
        ex_v[pl.ds(o, 16)] = ex
        plsc.addupdate_scatter(den_local, [d16], ex)
        return 0
    lax.fori_loop(0, VPW, _edge, 0)

    pltpu.sync_copy(ex_v.at[pl.ds(0, EPW)], ex_out.at[pl.ds(off, EPW)])
    pltpu.sync_copy(den_local, dens.at[s])
    plsc.subcore_barrier()

    for r in range(NS):
        pltpu.sync_copy(dens.at[r, pl.ds(r0, RPT)], dbuf.at[r])

    def _red(i, _):
        acc = dbuf[0, pl.ds(i * 16, 16)]
        for r in range(1, NS):
            acc = acc + dbuf[r, pl.ds(i * 16, 16)]
        d640[pl.ds(i * 16, 16)] = acc
        return 0
    lax.fori_loop(0, RPT // 16, _red, 0)
    pltpu.sync_copy(d640, dparts.at[c, pl.ds(r0, RPT)])


_sc_soft_call = functools.partial(
    pl.kernel,
    out_type=(jax.ShapeDtypeStruct((E,), jnp.float32),
              jax.ShapeDtypeStruct((NC, NPAD), jnp.float32)),
    mesh=plsc.VectorSubcoreMesh(
        core_axis_name="c", subcore_axis_name="s",
        num_cores=NC, num_subcores=NS),
    compiler_params=pltpu.CompilerParams(needs_layout_passes=False),
    scratch_types=[
        pltpu.VMEM_SHARED((NS, NPAD), jnp.float32),
        pltpu.VMEM((NPAD,), jnp.float32),
        pltpu.VMEM((NPAD,), jnp.float32),
        pltpu.VMEM((NPAD,), jnp.float32),
        pltpu.VMEM((NS, RPT), jnp.float32),
        pltpu.VMEM((EPAD,), jnp.int32),
        pltpu.VMEM((EPAD,), jnp.int32),
        pltpu.VMEM((EPAD,), jnp.float32),
        pltpu.VMEM((RPT,), jnp.float32),
    ],
)(_sc_soft)


def _sc_agg(xs2h, srch, dsth, exh, dparts, bias, out,
            ysum, rows, didx, src0, src1, dst0, dst1, ex0, ex1,
            dpb0, dpb1, dsum, bias_v, semL0, semL1, semG0, semG1, semS0, semS1):
    c = lax.axis_index("c")
    s = lax.axis_index("s")
    z16 = jnp.zeros((16,), jnp.float32)
    r0 = s * RPT
    srcb, dstb, exb = (src0, src1), (dst0, dst1), (ex0, ex1)
    semL, semG, semS = (semL0, semL1), (semG0, semG1), (semS0, semS1)

    pltpu.sync_copy(bias.at[pl.ds(c * FH, FH)], bias_v)

    def _z_rows(j, _):
        for m in range(FH // 16):
            rows[j, pl.ds(m * 16, 16)] = z16
        return 0
    lax.fori_loop(0, C, _z_rows, 0)

    pltpu.sync_copy(rows.at[pl.ds(0, RPT // 2)], ysum.at[pl.ds(r0, RPT // 2)])
    pltpu.sync_copy(rows.at[pl.ds(0, RPT // 2)],
                    ysum.at[pl.ds(r0 + RPT // 2, RPT // 2)])
    plsc.subcore_barrier()

    def _start_linears(k, b):
        off = s * EPT + jnp.minimum(k, NPC - 1) * PC
        pltpu.make_async_copy(srch.at[pl.ds(off, PC)], srcb[b], semL[b]).start()
        pltpu.make_async_copy(dsth.at[pl.ds(off, PC)], dstb[b], semL[b]).start()
        pltpu.make_async_copy(exh.at[pl.ds(off, PC)],
                              exb[b].at[pl.ds(0, PC)], semL[b]).start()

    def _wait_linears(b):
        pltpu.make_async_copy(srch.at[pl.ds(0, PC)], srcb[b], semL[b]).wait()
        pltpu.make_async_copy(dsth.at[pl.ds(0, PC)], dstb[b], semL[b]).wait()
        pltpu.make_async_copy(exh.at[pl.ds(0, PC)],
                              exb[b].at[pl.ds(0, PC)], semL[b]).wait()

    def _build_didx(b):
        for qa in range(PC // SUB):
            for qb in range(SUB // 16):
                didx[2 * b + qa, pl.ds(qb * 16, 16)] = (
                    dstb[b][pl.ds(qa * SUB + qb * 16, 16)])

    def _fire_gathers(b):
        for q in range(PC // 16):
            sidx = srcb[b][pl.ds(q * 16, 16)] * 2 + c
            pltpu.make_async_copy(
                xs2h.at[sidx], rows.at[pl.ds(b * PC + q * 16, 16)],
                semG[b]).start()

    def _wait_gathers(b):
        for q in range(PC // 16):
            sidx = srcb[b][pl.ds(q * 16, 16)] * 2 + c
            pltpu.make_async_copy(
                xs2h.at[sidx], rows.at[pl.ds(b * PC + q * 16, 16)],
                semG[b]).wait()

    def _scale(b):
        def _sj(j, _):
            al = exb[b][pl.ds(j, 16)][0]
            for m in range(FH // 16):
                rows[b * PC + j, pl.ds(m * 16, 16)] = (
                    rows[b * PC + j, pl.ds(m * 16, 16)] * al)
            return 0
        lax.fori_loop(0, PC, _sj, 0)

    def _scatter_desc(b, qa):
        return pltpu.make_async_copy(
            rows.at[pl.ds(b * PC + qa * SUB, SUB)],
            ysum.at[didx.at[2 * b + qa]], semS[b])

    def _step(k, b, first, last):
        if not first:
            for qa in range(PC // SUB):
                _scatter_desc(1 - b, qa).wait()
        _wait_linears(1 - b)
        _build_didx(1 - b)
        _fire_gathers(1 - b)
        _wait_gathers(b)
        _scale(b)
        if not last:
            _start_linears(k + 2, b)
        for qa in range(PC // SUB):
            _scatter_desc(b, qa).start(add=True)

    _start_linears(0, 0)
    _start_linears(1, 1)
    _wait_linears(0)
    _build_didx(0)
    _fire_gathers(0)
    _step(0, 0, True, False)

    def _pair(t, _):
        k = 1 + 2 * t
        _step(k, 1, False, False)
        _step(k + 1, 0, False, False)
        return 0
    lax.fori_loop(0, (NPC - 2) // 2, _pair, 0)

    for qa in range(PC // SUB):
        _scatter_desc(0, qa).wait()
    _wait_linears(0)
    _wait_gathers(1)
    _scale(1)
    for qa in range(PC // SUB):
        _scatter_desc(1, qa).start(add=True)
    for qa in range(PC // SUB):
        _scatter_desc(1, qa).wait()

    toff = s * EPT + NPC * PC
    pltpu.sync_copy(srch.at[pl.ds(toff, CT)], src0.at[pl.ds(0, CT)])
    pltpu.sync_copy(dsth.at[pl.ds(toff, CT)], dst0.at[pl.ds(0, CT)])
    pltpu.sync_copy(exh.at[pl.ds(toff, CT)], ex0.at[pl.ds(0, CT)])
    for qb in range(SUB // 16):
        didx[0, pl.ds(qb * 16, 16)] = dst0[pl.ds(qb * 16, 16)]
    tdescs = []
    for q in range(CT // 16):
        sidx = src0[pl.ds(q * 16, 16)] * 2 + c
        tdescs.append(pltpu.async_copy(
            xs2h.at[sidx], rows.at[pl.ds(q * 16, 16)], semG0))
    for dd in tdescs:
        dd.wait()

    def _tj(j, _):
        al = ex0[pl.ds(j, 16)][0]
        for m in range(FH // 16):
            rows[j, pl.ds(m * 16, 16)] = rows[j, pl.ds(m * 16, 16)] * al
        return 0
    lax.fori_loop(0, CT, _tj, 0)
    pltpu.sync_copy(rows.at[pl.ds(0, CT)], ysum.at[didx.at[0]], add=True)

    plsc.subcore_barrier()

    pltpu.sync_copy(dparts.at[0, pl.ds(r0, RPT)], dpb0)
    pltpu.sync_copy(dparts.at[1, pl.ds(r0, RPT)], dpb1)

    def _rcp(i, _):
        p = dpb0[pl.ds(i * 16, 16)] + dpb1[pl.ds(i * 16, 16)]
        dsum[pl.ds(i * 16, 16)] = 1.0 / (p + 1e-16)
        return 0
    lax.fori_loop(0, RPT // 16, _rcp, 0)

    for f in range(RPT // FIN):
        rr = r0 + f * FIN
        pltpu.sync_copy(ysum.at[pl.ds(rr, FIN)], rows.at[pl.ds(0, FIN)])

        def _frow(j, _):
            sc = dsum[pl.ds(f * FIN + j, 16)][0]
            for m in range(FH // 16):
                rows[j, pl.ds(m * 16, 16)] = (
                    rows[j, pl.ds(m * 16, 16)] * sc + bias_v[pl.ds(m * 16, 16)])
            return 0
        lax.fori_loop(0, FIN, _frow, 0)
        pltpu.sync_copy(rows.at[pl.ds(0, FIN)], out.at[c, pl.ds(rr, FIN)])


_sc_agg_call = functools.partial(
    pl.kernel,
    out_type=jax.ShapeDtypeStruct((NC, NPAD, FH), jnp.float32),
    mesh=plsc.VectorSubcoreMesh(
        core_axis_name="c", subcore_axis_name="s",
        num_cores=NC, num_subcores=NS),
    compiler_params=pltpu.CompilerParams(needs_layout_passes=False),
    scratch_types=[
        pltpu.VMEM_SHARED((NPAD, FH), jnp.float32),
        pltpu.VMEM((C, FH), jnp.float32),
        pltpu.VMEM((C // SUB, SUB), jnp.int32),
        pltpu.VMEM((PC,), jnp.int32),
        pltpu.VMEM((PC,), jnp.int32),
        pltpu.VMEM((PC,), jnp.int32),
        pltpu.VMEM((PC,), jnp.int32),
        pltpu.VMEM((PC + 16,), jnp.float32),
        pltpu.VMEM((PC + 16,), jnp.float32),
        pltpu.VMEM((RPT,), jnp.float32),
        pltpu.VMEM((RPT,), jnp.float32),
        pltpu.VMEM((RPT + 16,), jnp.float32),
        pltpu.VMEM((FH,), jnp.float32),
        pltpu.SemaphoreType.DMA,
        pltpu.SemaphoreType.DMA,
        pltpu.SemaphoreType.DMA,
        pltpu.SemaphoreType.DMA,
        pltpu.SemaphoreType.DMA,
        pltpu.SemaphoreType.DMA,
    ],
)(_sc_agg)


def _tc_layer1(x_pad, W1s, W1d, a1s, a1d):
    return pl.pallas_call(
        _tc_first,
        grid=(NPAD // 1024,),
        in_specs=[
            pl.BlockSpec((1024, D), lambda i: (i, 0)),
            pl.BlockSpec((D, F), lambda i: (0, 0)),
            pl.BlockSpec((D, F), lambda i: (0, 0)),
            pl.BlockSpec((1, F), lambda i: (0, 0)),
            pl.BlockSpec((1, F), lambda i: (0, 0)),
        ],
        out_specs=[
            pl.BlockSpec((1024, F), lambda i: (i, 0)),
            pl.BlockSpec((NPAD // 1024, 1024), lambda i: (0, 0)),
            pl.BlockSpec((NPAD // 1024, 1024), lambda i: (0, 0)),
        ],
        out_shape=[
            jax.ShapeDtypeStruct((NPAD, F), jnp.float32),
            jax.ShapeDtypeStruct((NPAD // 1024, 1024), jnp.float32),
            jax.ShapeDtypeStruct((NPAD // 1024, 1024), jnp.float32),
        ],
    )(x_pad, W1s, W1d, a1s.reshape(1, F), a1d.reshape(1, F))


def _tc_layer2(h, W2s, W2d, a2s, a2d):
    return pl.pallas_call(
        _tc_second,
        grid=(NPAD // 1024,),
        in_specs=[
            pl.BlockSpec((NC, 1024, FH), lambda i: (0, i, 0)),
            pl.BlockSpec((F, F), lambda i: (0, 0)),
            pl.BlockSpec((F, F), lambda i: (0, 0)),
            pl.BlockSpec((1, F), lambda i: (0, 0)),
            pl.BlockSpec((1, F), lambda i: (0, 0)),
        ],
        out_specs=[
            pl.BlockSpec((1024, F), lambda i: (i, 0)),
            pl.BlockSpec((NPAD // 1024, 1024), lambda i: (0, 0)),
            pl.BlockSpec((NPAD // 1024, 1024), lambda i: (0, 0)),
        ],
        out_shape=[
            jax.ShapeDtypeStruct((NPAD, F), jnp.float32),
            jax.ShapeDtypeStruct((NPAD // 1024, 1024), jnp.float32),
            jax.ShapeDtypeStruct((NPAD // 1024, 1024), jnp.float32),
        ],
    )(h, W2s, W2d, a2s.reshape(1, F), a2d.reshape(1, F))


def kernel(x, edge_index, W1s, W1d, a1s, a1d, b1, W2s, W2d, a2s, a2d, b2):
    x_pad = jnp.pad(x, ((0, NPAD - N), (0, 0)))
    src = edge_index[0]
    dst = edge_index[1]

    xs1, asrc1, adst1 = _tc_layer1(x_pad, W1s, W1d, a1s, a1d)
    ex1, dparts1 = _sc_soft_call(asrc1.reshape(NPAD), adst1.reshape(NPAD),
                                 src, dst)
    h = _sc_agg_call(xs1.reshape(2 * NPAD, FH), src, dst, ex1, dparts1, b1)

    xs2, asrc2, adst2 = _tc_layer2(h, W2s, W2d, a2s, a2d)
    ex2, dparts2 = _sc_soft_call(asrc2.reshape(NPAD), adst2.reshape(NPAD),
                                 src, dst)
    out2 = _sc_agg_call(xs2.reshape(2 * NPAD, FH), src, dst, ex2, dparts2, b2)

    return out2.transpose(1, 0, 2).reshape(NPAD, F)[:N]


# speedup vs baseline: 19.6763x; 1.1303x over previous
import functools

import jax
import jax.numpy as jnp
from jax import lax
from jax.experimental import pallas as pl
from jax.experimental.pallas import tpu as pltpu
from jax.experimental.pallas import tpu_sc as plsc

N = 10000
NPAD = 10240
E = 160000
D = 256
F = 256
FH = 128
NEG_SLOPE = 0.2

NC = 2
NS = 16
RPT = NPAD // NS

EPW = E // (NC * NS)
VPW = 313
EPAD = VPW * 16

EPT = E // NS
PC = 128
C = 2 * PC
NPC = EPT // PC
CT = EPT - NPC * PC
FIN = 128
NFIN = RPT // FIN


def _tc_first(x_ref, ws_ref, wd_ref, as_ref, ad_ref, xs_ref, asrc_ref, adst_ref):
    i = pl.program_id(0)
    xb = x_ref[...]
    xs = jnp.dot(xb, ws_ref[...], preferred_element_type=jnp.float32)
    xs_ref[...] = xs
    nb = xb.shape[0]
    asrc_ref[pl.ds(i, 1), :] = jnp.sum(xs * as_ref[...], axis=1).reshape(1, nb)
    wdv = jnp.sum(wd_ref[...] * ad_ref[...], axis=1)
    adst_ref[pl.ds(i, 1), :] = jnp.sum(xb * wdv[None, :], axis=1).reshape(1, nb)


def _tc_second(h_ref, ws_ref, wd_ref, as_ref, ad_ref, xs_ref, asrc_ref, adst_ref):
    i = pl.program_id(0)
    hb = jax.nn.relu(h_ref[...])
    xs = jnp.dot(hb, ws_ref[...], preferred_element_type=jnp.float32)
    xs_ref[...] = xs
    nb = hb.shape[0]
    asrc_ref[pl.ds(i, 1), :] = jnp.sum(xs * as_ref[...], axis=1).reshape(1, nb)
    wdv = jnp.sum(wd_ref[...] * ad_ref[...], axis=1)
    adst_ref[pl.ds(i, 1), :] = jnp.sum(hb * wdv[None, :], axis=1).reshape(1, nb)


def _sc_soft(asrc, adst, srch, dsth, ex_out, dparts,
             dens, asrc_v, adst_v, den_local, dbuf, src_v, dst_v, ex_v, d640):
    c = lax.axis_index("c")
    s = lax.axis_index("s")
    wid = c * NS + s
    off = wid * EPW
    r0 = s * RPT
    z16 = jnp.zeros((16,), jnp.float32)
    iota = lax.iota(jnp.int32, 16)
    nt = asrc.shape[0]

    pltpu.sync_copy(asrc, asrc_v.at[pl.ds(0, nt)])
    pltpu.sync_copy(adst, adst_v.at[pl.ds(0, nt)])
    pltpu.sync_copy(srch.at[pl.ds(off, EPW)], src_v.at[pl.ds(0, EPW)])
    pltpu.sync_copy(dsth.at[pl.ds(off, EPW)], dst_v.at[pl.ds(0, EPW)])

    def _z_den(i, _):
        den_local[pl.ds(i * 16, 16)] = z16
        return 0
    lax.fori_loop(0, NPAD // 16, _z_den, 0)

    tl = EPAD - 16
    tv = src_v[pl.ds(tl, 16)]
    src_v[pl.ds(tl, 16)] = jnp.where(iota < EPW - tl, tv, 0)
    tv = dst_v[pl.ds(tl, 16)]
    dst_v[pl.ds(tl, 16)] = jnp.where(iota < EPW - tl, tv, 0)

    def _edge(i, _):
        o = i * 16
        s16 = src_v[pl.ds(o, 16)]
        d16 = dst_v[pl.ds(o, 16)]
        av = plsc.load_gather(asrc_v, [s16])
        dv = plsc.load_gather(adst_v, [d16])
        e = av + dv
        e = jnp.where(e > 0, e, e * NEG_SLOPE)
        ex = jnp.exp(e)
        ex = jnp.where(o + iota < EPW, ex, 0.0)
        ex_v[pl.ds(o, 16)] = ex
        plsc.addupdate_scatter(den_local, [d16], ex)
        return 0
    lax.fori_loop(0, VPW, _edge, 0)

    pltpu.sync_copy(ex_v.at[pl.ds(0, EPW)], ex_out.at[pl.ds(off, EPW)])
    pltpu.sync_copy(den_local, dens.at[s])
    plsc.subcore_barrier()

    for r in range(NS):
        pltpu.sync_copy(dens.at[r, pl.ds(r0, RPT)], dbuf.at[r])

    def _red(i, _):
        acc = dbuf[0, pl.ds(i * 16, 16)]
        for r in range(1, NS):
            acc = acc + dbuf[r, pl.ds(i * 16, 16)]
        d640[pl.ds(i * 16, 16)] = acc
        return 0
    lax.fori_loop(0, RPT // 16, _red, 0)
    pltpu.sync_copy(d640, dparts.at[c, pl.ds(r0, RPT)])


_sc_soft_call = functools.partial(
    pl.kernel,
    out_type=(jax.ShapeDtypeStruct((E,), jnp.float32),
              jax.ShapeDtypeStruct((NC, NPAD), jnp.float32)),
    mesh=plsc.VectorSubcoreMesh(
        core_axis_name="c", subcore_axis_name="s",
        num_cores=NC, num_subcores=NS),
    compiler_params=pltpu.CompilerParams(needs_layout_passes=False),
    scratch_types=[
        pltpu.VMEM_SHARED((NS, NPAD), jnp.float32),
        pltpu.VMEM((NPAD,), jnp.float32),
        pltpu.VMEM((NPAD,), jnp.float32),
        pltpu.VMEM((NPAD,), jnp.float32),
        pltpu.VMEM((NS, RPT), jnp.float32),
        pltpu.VMEM((EPAD,), jnp.int32),
        pltpu.VMEM((EPAD,), jnp.int32),
        pltpu.VMEM((EPAD,), jnp.float32),
        pltpu.VMEM((RPT,), jnp.float32),
    ],
)(_sc_soft)


def _sc_agg(xs2h, srch, dsth, exh, dparts, bias, out,
            ysum, rows, didx, gidx, src0, src1, dst0, dst1, ex0, ex1,
            dpb0, dpb1, dsum, bias_v, semL0, semL1, semG0, semG1, semS0, semS1):
    c = lax.axis_index("c")
    s = lax.axis_index("s")
    z16 = jnp.zeros((16,), jnp.float32)
    iota = lax.iota(jnp.int32, 16)
    r0 = s * RPT
    srcb, dstb, exb = (src0, src1), (dst0, dst1), (ex0, ex1)
    semL, semG, semS = (semL0, semL1), (semG0, semG1), (semS0, semS1)

    pltpu.sync_copy(bias.at[pl.ds(c * FH, FH)], bias_v)

    def _z_rows(j, _):
        for m in range(FH // 16):
            rows[j, pl.ds(m * 16, 16)] = z16
        return 0
    lax.fori_loop(0, C, _z_rows, 0)

    pltpu.sync_copy(rows.at[pl.ds(0, C)], ysum.at[pl.ds(r0, C)])
    pltpu.sync_copy(rows.at[pl.ds(0, C)], ysum.at[pl.ds(r0 + C, C)])
    pltpu.sync_copy(rows.at[pl.ds(0, RPT - 2 * C)],
                    ysum.at[pl.ds(r0 + 2 * C, RPT - 2 * C)])
    plsc.subcore_barrier()

    def _start_linears(k, b):
        off = s * EPT + jnp.minimum(k, NPC - 1) * PC
        pltpu.make_async_copy(srch.at[pl.ds(off, PC)], srcb[b], semL[b]).start()
        pltpu.make_async_copy(dsth.at[pl.ds(off, PC)], dstb[b], semL[b]).start()
        pltpu.make_async_copy(exh.at[pl.ds(off, PC)],
                              exb[b].at[pl.ds(0, PC)], semL[b]).start()

    def _wait_linears(b):
        pltpu.make_async_copy(srch.at[pl.ds(0, PC)], srcb[b], semL[b]).wait()
        pltpu.make_async_copy(dsth.at[pl.ds(0, PC)], dstb[b], semL[b]).wait()
        pltpu.make_async_copy(exh.at[pl.ds(0, PC)],
                              exb[b].at[pl.ds(0, PC)], semL[b]).wait()

    def _build_idx(b):
        for q in range(PC // 16):
            didx[b, pl.ds(q * 16, 16)] = dstb[b][pl.ds(q * 16, 16)]
            gidx[b, pl.ds(q * 16, 16)] = srcb[b][pl.ds(q * 16, 16)] * 2 + c

    def _gather_desc(b):
        return pltpu.make_async_copy(
            xs2h.at[gidx.at[b]], rows.at[pl.ds(b * PC, PC)], semG[b])

    def _scatter_desc(b):
        return pltpu.make_async_copy(
            rows.at[pl.ds(b * PC, PC)], ysum.at[didx.at[b]], semS[b])

    def _scale(b):
        def _sj(j, _):
            j4 = j * 4
            for u in range(4):
                al = exb[b][pl.ds(j4 + u, 16)][0]
                for m in range(FH // 16):
                    rows[b * PC + j4 + u, pl.ds(m * 16, 16)] = (
                        rows[b * PC + j4 + u, pl.ds(m * 16, 16)] * al)
            return 0
        lax.fori_loop(0, PC // 4, _sj, 0)

    def _step(k, b, first, last):
        if not first:
            _scatter_desc(1 - b).wait()
        _wait_linears(1 - b)
        _build_idx(1 - b)
        _gather_desc(1 - b).start()
        _gather_desc(b).wait()
        _scale(b)
        if not last:
            _start_linears(k + 2, b)
        _scatter_desc(b).start(add=True)

    _start_linears(0, 0)
    _start_linears(1, 1)
    _wait_linears(0)
    _build_idx(0)
    _gather_desc(0).start()
    _step(0, 0, True, False)

    def _pair(t, _):
        k = 1 + 2 * t
        _step(k, 1, False, False)
        _step(k + 1, 0, False, False)
        return 0
    lax.fori_loop(0, (NPC - 2) // 2, _pair, 0)

    _scatter_desc(0).wait()
    _wait_linears(0)
    _gather_desc(1).wait()
    _scale(1)
    _scatter_desc(1).start(add=True)
    _scatter_desc(1).wait()

    toff = s * EPT + NPC * PC
    pltpu.sync_copy(srch.at[pl.ds(toff, CT)], src0.at[pl.ds(0, CT)])
    pltpu.sync_copy(dsth.at[pl.ds(toff, CT)], dst0.at[pl.ds(0, CT)])
    pltpu.sync_copy(exh.at[pl.ds(toff, CT)], ex0.at[pl.ds(0, CT)])
    s16 = src0[pl.ds(0, 16)] * 2 + c
    d16 = dst0[pl.ds(0, 16)]
    pltpu.make_async_copy(xs2h.at[s16], rows.at[pl.ds(0, 16)], semG0).start()
    pltpu.make_async_copy(xs2h.at[s16], rows.at[pl.ds(0, 16)], semG0).wait()

    def _tj(j, _):
        al = ex0[pl.ds(j, 16)][0]
        for m in range(FH // 16):
            rows[j, pl.ds(m * 16, 16)] = rows[j, pl.ds(m * 16, 16)] * al
        return 0
    lax.fori_loop(0, CT, _tj, 0)
    pltpu.make_async_copy(rows.at[pl.ds(0, 16)], ysum.at[d16], semS0
                          ).start(add=True)
    pltpu.make_async_copy(rows.at[pl.ds(0, 16)], ysum.at[d16], semS0).wait()

    plsc.subcore_barrier()

    pltpu.sync_copy(dparts.at[0, pl.ds(r0, RPT)], dpb0)
    pltpu.sync_copy(dparts.at[1, pl.ds(r0, RPT)], dpb1)

    def _rcp(i, _):
        p = dpb0[pl.ds(i * 16, 16)] + dpb1[pl.ds(i * 16, 16)]
        dsum[pl.ds(i * 16, 16)] = 1.0 / (p + 1e-16)
        return 0
    lax.fori_loop(0, RPT // 16, _rcp, 0)

    bvs = [bias_v[pl.ds(m * 16, 16)] for m in range(FH // 16)]
    for f in range(NFIN):
        rr = r0 + f * FIN
        pltpu.sync_copy(ysum.at[pl.ds(rr, FIN)], rows.at[pl.ds(0, FIN)])
        for q in range(FIN // 16):
            gidx[0, pl.ds(q * 16, 16)] = (rr + q * 16 + iota) * 2 + c

        def _frow(j, _):
            sc = dsum[pl.ds(f * FIN + j, 16)][0]
            for m in range(FH // 16):
                rows[j, pl.ds(m * 16, 16)] = (
                    rows[j, pl.ds(m * 16, 16)] * sc + bvs[m])
            return 0
        lax.fori_loop(0, FIN, _frow, 0)
        pltpu.sync_copy(rows.at[pl.ds(0, FIN)], out.at[gidx.at[0]])


_sc_agg_call = functools.partial(
    pl.kernel,
    out_type=jax.ShapeDtypeStruct((2 * NPAD, FH), jnp.float32),
    mesh=plsc.VectorSubcoreMesh(
        core_axis_name="c", subcore_axis_name="s",
        num_cores=NC, num_subcores=NS),
    compiler_params=pltpu.CompilerParams(needs_layout_passes=False),
    scratch_types=[
        pltpu.VMEM_SHARED((NPAD, FH), jnp.float32),
        pltpu.VMEM((C, FH), jnp.float32),
        pltpu.VMEM((2, PC), jnp.int32),
        pltpu.VMEM((2, PC), jnp.int32),
        pltpu.VMEM((PC,), jnp.int32),
        pltpu.VMEM((PC,), jnp.int32),
        pltpu.VMEM((PC,), jnp.int32),
        pltpu.VMEM((PC,), jnp.int32),
        pltpu.VMEM((PC + 16,), jnp.float32),
        pltpu.VMEM((PC + 16,), jnp.float32),
        pltpu.VMEM((RPT,), jnp.float32),
        pltpu.VMEM((RPT,), jnp.float32),
        pltpu.VMEM((RPT + 16,), jnp.float32),
        pltpu.VMEM((FH,), jnp.float32),
        pltpu.SemaphoreType.DMA,
        pltpu.SemaphoreType.DMA,
        pltpu.SemaphoreType.DMA,
        pltpu.SemaphoreType.DMA,
        pltpu.SemaphoreType.DMA,
        pltpu.SemaphoreType.DMA,
    ],
)(_sc_agg)


def _tc_layer(body, h, Ws, Wd, a_s, a_d):
    nb = h.shape[0] // 10
    return pl.pallas_call(
        body,
        grid=(10,),
        in_specs=[
            pl.BlockSpec((nb, D), lambda i: (i, 0)),
            pl.BlockSpec((D, F), lambda i: (0, 0)),
            pl.BlockSpec((D, F), lambda i: (0, 0)),
            pl.BlockSpec((1, F), lambda i: (0, 0)),
            pl.BlockSpec((1, F), lambda i: (0, 0)),
        ],
        out_specs=[
            pl.BlockSpec((nb, F), lambda i: (i, 0)),
            pl.BlockSpec((10, nb), lambda i: (0, 0)),
            pl.BlockSpec((10, nb), lambda i: (0, 0)),
        ],
        out_shape=[
            jax.ShapeDtypeStruct((h.shape[0], F), jnp.float32),
            jax.ShapeDtypeStruct((10, nb), jnp.float32),
            jax.ShapeDtypeStruct((10, nb), jnp.float32),
        ],
    )(h, Ws, Wd, a_s.reshape(1, F), a_d.reshape(1, F))


def kernel(x, edge_index, W1s, W1d, a1s, a1d, b1, W2s, W2d, a2s, a2d, b2):
    src = edge_index[0]
    dst = edge_index[1]

    xs1, asrc1, adst1 = _tc_layer(_tc_first, x, W1s, W1d, a1s, a1d)
    ex1, dparts1 = _sc_soft_call(asrc1.reshape(N), adst1.reshape(N), src, dst)
    h2i = _sc_agg_call(xs1.reshape(2 * N, FH), src, dst, ex1, dparts1, b1)
    h = h2i.reshape(NPAD, F)

    xs2, asrc2, adst2 = _tc_layer(_tc_second, h, W2s, W2d, a2s, a2d)
    ex2, dparts2 = _sc_soft_call(asrc2.reshape(NPAD), adst2.reshape(NPAD),
                                 src, dst)
    out2 = _sc_agg_call(xs2.reshape(2 * NPAD, FH), src, dst, ex2, dparts2, b2)

    return out2.reshape(NPAD, F)[:N]


# speedup vs baseline: 22.7907x; 1.1583x over previous
import functools

import jax
import jax.numpy as jnp
from jax import lax
from jax.experimental import pallas as pl
from jax.experimental.pallas import tpu as pltpu
from jax.experimental.pallas import tpu_sc as plsc

N = 10000
NPAD = 10240
E = 160000
D = 256
F = 256
FH = 128
NEG_SLOPE = 0.2

NC = 2
NS = 16
RPT = NPAD // NS

EPW = E // (NC * NS)
VPW = 313
EPAD = VPW * 16

EPT = E // NS
PC = 128
C = 2 * PC
NPC = EPT // PC
CT = EPT - NPC * PC
FIN = 128
NFIN = RPT // FIN


def _tc_first(x_ref, ws_ref, wd_ref, as_ref, ad_ref, xs_ref, asrc_ref, adst_ref):
    i = pl.program_id(0)
    xb = x_ref[...]
    xs = jnp.dot(xb, ws_ref[...], preferred_element_type=jnp.float32)
    xs_ref[...] = xs
    nb = xb.shape[0]
    asrc_ref[pl.ds(i, 1), :] = jnp.sum(xs * as_ref[...], axis=1).reshape(1, nb)
    wdv = jnp.sum(wd_ref[...] * ad_ref[...], axis=1)
    adst_ref[pl.ds(i, 1), :] = jnp.sum(xb * wdv[None, :], axis=1).reshape(1, nb)


def _tc_second(h_ref, ws_ref, wd_ref, as_ref, ad_ref, xs_ref, asrc_ref, adst_ref):
    i = pl.program_id(0)
    hb = jax.nn.relu(h_ref[...])
    xs = jnp.dot(hb, ws_ref[...], preferred_element_type=jnp.float32)
    xs_ref[...] = xs
    nb = hb.shape[0]
    asrc_ref[pl.ds(i, 1), :] = jnp.sum(xs * as_ref[...], axis=1).reshape(1, nb)
    wdv = jnp.sum(wd_ref[...] * ad_ref[...], axis=1)
    adst_ref[pl.ds(i, 1), :] = jnp.sum(hb * wdv[None, :], axis=1).reshape(1, nb)


def _sc_soft(asrc, adst, srch, dsth, ex_out, dparts,
             dens, asrc_v, adst_v, den_local, dbuf, src_v, dst_v, ex_v, d640):
    c = lax.axis_index("c")
    s = lax.axis_index("s")
    wid = c * NS + s
    off = wid * EPW
    r0 = s * RPT
    z16 = jnp.zeros((16,), jnp.float32)
    iota = lax.iota(jnp.int32, 16)
    nt = asrc.shape[0]

    pltpu.sync_copy(asrc, asrc_v.at[pl.ds(0, nt)])
    pltpu.sync_copy(adst, adst_v.at[pl.ds(0, nt)])
    pltpu.sync_copy(srch.at[pl.ds(off, EPW)], src_v.at[pl.ds(0, EPW)])
    pltpu.sync_copy(dsth.at[pl.ds(off, EPW)], dst_v.at[pl.ds(0, EPW)])

    def _z_den(i, _):
        den_local[pl.ds(i * 16, 16)] = z16
        return 0
    lax.fori_loop(0, NPAD // 16, _z_den, 0)

    tl = EPAD - 16
    tv = src_v[pl.ds(tl, 16)]
    src_v[pl.ds(tl, 16)] = jnp.where(iota < EPW - tl, tv, 0)
    tv = dst_v[pl.ds(tl, 16)]
    dst_v[pl.ds(tl, 16)] = jnp.where(iota < EPW - tl, tv, 0)

    def _edge(i, _):
        o = i * 16
        s16 = src_v[pl.ds(o, 16)]
        d16 = dst_v[pl.ds(o, 16)]
        av = plsc.load_gather(asrc_v, [s16])
        dv = plsc.load_gather(adst_v, [d16])
        e = av + dv
        e = jnp.where(e > 0, e, e * NEG_SLOPE)
        ex = jnp.exp(e)
        ex = jnp.where(o + iota < EPW, ex, 0.0)
        ex_v[pl.ds(o, 16)] = ex
        plsc.addupdate_scatter(den_local, [d16], ex)
        return 0
    lax.fori_loop(0, VPW, _edge, 0)

    pltpu.sync_copy(ex_v.at[pl.ds(0, EPW)], ex_out.at[pl.ds(off, EPW)])
    pltpu.sync_copy(den_local, dens.at[s])
    plsc.subcore_barrier()

    for r in range(NS):
        pltpu.sync_copy(dens.at[r, pl.ds(r0, RPT)], dbuf.at[r])

    def _red(i, _):
        acc = dbuf[0, pl.ds(i * 16, 16)]
        for r in range(1, NS):
            acc = acc + dbuf[r, pl.ds(i * 16, 16)]
        d640[pl.ds(i * 16, 16)] = acc
        return 0
    lax.fori_loop(0, RPT // 16, _red, 0)
    pltpu.sync_copy(d640, dparts.at[c, pl.ds(r0, RPT)])


_sc_soft_call = functools.partial(
    pl.kernel,
    out_type=(jax.ShapeDtypeStruct((E,), jnp.float32),
              jax.ShapeDtypeStruct((NC, NPAD), jnp.float32)),
    mesh=plsc.VectorSubcoreMesh(
        core_axis_name="c", subcore_axis_name="s",
        num_cores=NC, num_subcores=NS),
    compiler_params=pltpu.CompilerParams(needs_layout_passes=False),
    scratch_types=[
        pltpu.VMEM_SHARED((NS, NPAD), jnp.float32),
        pltpu.VMEM((NPAD,), jnp.float32),
        pltpu.VMEM((NPAD,), jnp.float32),
        pltpu.VMEM((NPAD,), jnp.float32),
        pltpu.VMEM((NS, RPT), jnp.float32),
        pltpu.VMEM((EPAD,), jnp.int32),
        pltpu.VMEM((EPAD,), jnp.int32),
        pltpu.VMEM((EPAD,), jnp.float32),
        pltpu.VMEM((RPT,), jnp.float32),
    ],
)(_sc_soft)


def _sc_agg(xs2h, srch, dsth, exh, dparts, bias, out,
            ysum, rows, didx, gidx, src0, src1, dst0, dst1, ex0, ex1,
            dpb0, dpb1, dsum, bias_v, semL0, semL1, semG0, semG1, semS0, semS1):
    c = lax.axis_index("c")
    s = lax.axis_index("s")
    z16 = jnp.zeros((16,), jnp.float32)
    iota = lax.iota(jnp.int32, 16)
    r0 = s * RPT
    srcb, dstb, exb = (src0, src1), (dst0, dst1), (ex0, ex1)
    semL, semG, semS = (semL0, semL1), (semG0, semG1), (semS0, semS1)

    pltpu.sync_copy(bias.at[pl.ds(c * FH, FH)], bias_v)

    def _z_rows(j, _):
        for m in range(FH // 16):
            rows[j, pl.ds(m * 16, 16)] = z16
        return 0
    lax.fori_loop(0, C, _z_rows, 0)

    pltpu.sync_copy(rows.at[pl.ds(0, C)], ysum.at[pl.ds(r0, C)])
    pltpu.sync_copy(rows.at[pl.ds(0, C)], ysum.at[pl.ds(r0 + C, C)])
    pltpu.sync_copy(rows.at[pl.ds(0, RPT - 2 * C)],
                    ysum.at[pl.ds(r0 + 2 * C, RPT - 2 * C)])
    plsc.subcore_barrier()

    def _start_linears(k, b):
        off = s * EPT + jnp.minimum(k, NPC - 1) * PC
        pltpu.make_async_copy(srch.at[pl.ds(off, PC)], srcb[b], semL[b]).start()
        pltpu.make_async_copy(dsth.at[pl.ds(off, PC)], dstb[b], semL[b]).start()
        pltpu.make_async_copy(exh.at[pl.ds(off, PC)],
                              exb[b].at[pl.ds(0, PC)], semL[b]).start()

    def _wait_linears(b):
        pltpu.make_async_copy(srch.at[pl.ds(0, PC)], srcb[b], semL[b]).wait()
        pltpu.make_async_copy(dsth.at[pl.ds(0, PC)], dstb[b], semL[b]).wait()
        pltpu.make_async_copy(exh.at[pl.ds(0, PC)],
                              exb[b].at[pl.ds(0, PC)], semL[b]).wait()

    def _build_idx(b):
        for q in range(PC // 16):
            didx[b, pl.ds(q * 16, 16)] = dstb[b][pl.ds(q * 16, 16)]
            gidx[b, pl.ds(q * 16, 16)] = srcb[b][pl.ds(q * 16, 16)] * 2 + c

    def _gather_desc(b):
        return pltpu.make_async_copy(
            xs2h.at[gidx.at[b]], rows.at[pl.ds(b * PC, PC)], semG[b])

    def _scatter_desc(b):
        return pltpu.make_async_copy(
            rows.at[pl.ds(b * PC, PC)], ysum.at[didx.at[b]], semS[b])

    def _scale(b):
        @plsc.parallel_loop(0, PC, step=1, unroll=4)
        def _sj(j):
            al = exb[b][pl.ds(j, 16)][0]
            for m in range(FH // 16):
                rows[b * PC + j, pl.ds(m * 16, 16)] = (
                    rows[b * PC + j, pl.ds(m * 16, 16)] * al)

    def _step(k, b, first, last):
        if not first:
            _scatter_desc(1 - b).wait()
        _wait_linears(1 - b)
        _build_idx(1 - b)
        _gather_desc(1 - b).start()
        _gather_desc(b).wait()
        _scale(b)
        if not last:
            _start_linears(k + 2, b)
        _scatter_desc(b).start(add=True)

    _start_linears(0, 0)
    _start_linears(1, 1)
    _wait_linears(0)
    _build_idx(0)
    _gather_desc(0).start()
    _step(0, 0, True, False)

    def _pair(t, _):
        k = 1 + 2 * t
        _step(k, 1, False, False)
        _step(k + 1, 0, False, False)
        return 0
    lax.fori_loop(0, (NPC - 2) // 2, _pair, 0)

    _scatter_desc(0).wait()
    _wait_linears(0)
    _gather_desc(1).wait()
    _scale(1)
    _scatter_desc(1).start(add=True)
    _scatter_desc(1).wait()

    toff = s * EPT + NPC * PC
    pltpu.sync_copy(srch.at[pl.ds(toff, CT)], src0.at[pl.ds(0, CT)])
    pltpu.sync_copy(dsth.at[pl.ds(toff, CT)], dst0.at[pl.ds(0, CT)])
    pltpu.sync_copy(exh.at[pl.ds(toff, CT)], ex0.at[pl.ds(0, CT)])
    s16 = src0[pl.ds(0, 16)] * 2 + c
    d16 = dst0[pl.ds(0, 16)]
    pltpu.make_async_copy(xs2h.at[s16], rows.at[pl.ds(0, 16)], semG0).start()
    pltpu.make_async_copy(xs2h.at[s16], rows.at[pl.ds(0, 16)], semG0).wait()

    def _tj(j, _):
        al = ex0[pl.ds(j, 16)][0]
        for m in range(FH // 16):
            rows[j, pl.ds(m * 16, 16)] = rows[j, pl.ds(m * 16, 16)] * al
        return 0
    lax.fori_loop(0, CT, _tj, 0)
    pltpu.make_async_copy(rows.at[pl.ds(0, 16)], ysum.at[d16], semS0
                          ).start(add=True)
    pltpu.make_async_copy(rows.at[pl.ds(0, 16)], ysum.at[d16], semS0).wait()

    plsc.subcore_barrier()

    pltpu.sync_copy(dparts.at[0, pl.ds(r0, RPT)], dpb0)
    pltpu.sync_copy(dparts.at[1, pl.ds(r0, RPT)], dpb1)

    def _rcp(i, _):
        p = dpb0[pl.ds(i * 16, 16)] + dpb1[pl.ds(i * 16, 16)]
        dsum[pl.ds(i * 16, 16)] = 1.0 / (p + 1e-16)
        return 0
    lax.fori_loop(0, RPT // 16, _rcp, 0)

    bvs = [bias_v[pl.ds(m * 16, 16)] for m in range(FH // 16)]
    for f in range(NFIN):
        rr = r0 + f * FIN
        pltpu.sync_copy(ysum.at[pl.ds(rr, FIN)], rows.at[pl.ds(0, FIN)])
        for q in range(FIN // 16):
            gidx[0, pl.ds(q * 16, 16)] = (rr + q * 16 + iota) * 2 + c

        @plsc.parallel_loop(0, FIN, step=1, unroll=4)
        def _frow(j):
            sc = dsum[pl.ds(f * FIN + j, 16)][0]
            for m in range(FH // 16):
                rows[j, pl.ds(m * 16, 16)] = (
                    rows[j, pl.ds(m * 16, 16)] * sc + bvs[m])
        pltpu.sync_copy(rows.at[pl.ds(0, FIN)], out.at[gidx.at[0]])


_sc_agg_call = functools.partial(
    pl.kernel,
    out_type=jax.ShapeDtypeStruct((2 * NPAD, FH), jnp.float32),
    mesh=plsc.VectorSubcoreMesh(
        core_axis_name="c", subcore_axis_name="s",
        num_cores=NC, num_subcores=NS),
    compiler_params=pltpu.CompilerParams(needs_layout_passes=False),
    scratch_types=[
        pltpu.VMEM_SHARED((NPAD, FH), jnp.float32),
        pltpu.VMEM((C, FH), jnp.float32),
        pltpu.VMEM((2, PC), jnp.int32),
        pltpu.VMEM((2, PC), jnp.int32),
        pltpu.VMEM((PC,), jnp.int32),
        pltpu.VMEM((PC,), jnp.int32),
        pltpu.VMEM((PC,), jnp.int32),
        pltpu.VMEM((PC,), jnp.int32),
        pltpu.VMEM((PC + 16,), jnp.float32),
        pltpu.VMEM((PC + 16,), jnp.float32),
        pltpu.VMEM((RPT,), jnp.float32),
        pltpu.VMEM((RPT,), jnp.float32),
        pltpu.VMEM((RPT + 16,), jnp.float32),
        pltpu.VMEM((FH,), jnp.float32),
        pltpu.SemaphoreType.DMA,
        pltpu.SemaphoreType.DMA,
        pltpu.SemaphoreType.DMA,
        pltpu.SemaphoreType.DMA,
        pltpu.SemaphoreType.DMA,
        pltpu.SemaphoreType.DMA,
    ],
)(_sc_agg)


def _tc_layer(body, h, Ws, Wd, a_s, a_d):
    nb = h.shape[0] // 10
    return pl.pallas_call(
        body,
        grid=(10,),
        in_specs=[
            pl.BlockSpec((nb, D), lambda i: (i, 0)),
            pl.BlockSpec((D, F), lambda i: (0, 0)),
            pl.BlockSpec((D, F), lambda i: (0, 0)),
            pl.BlockSpec((1, F), lambda i: (0, 0)),
            pl.BlockSpec((1, F), lambda i: (0, 0)),
        ],
        out_specs=[
            pl.BlockSpec((nb, F), lambda i: (i, 0)),
            pl.BlockSpec((10, nb), lambda i: (0, 0)),
            pl.BlockSpec((10, nb), lambda i: (0, 0)),
        ],
        out_shape=[
            jax.ShapeDtypeStruct((h.shape[0], F), jnp.float32),
            jax.ShapeDtypeStruct((10, nb), jnp.float32),
            jax.ShapeDtypeStruct((10, nb), jnp.float32),
        ],
    )(h, Ws, Wd, a_s.reshape(1, F), a_d.reshape(1, F))


def kernel(x, edge_index, W1s, W1d, a1s, a1d, b1, W2s, W2d, a2s, a2d, b2):
    src = edge_index[0]
    dst = edge_index[1]

    xs1, asrc1, adst1 = _tc_layer(_tc_first, x, W1s, W1d, a1s, a1d)
    ex1, dparts1 = _sc_soft_call(asrc1.reshape(N), adst1.reshape(N), src, dst)
    h2i = _sc_agg_call(xs1.reshape(2 * N, FH), src, dst, ex1, dparts1, b1)
    h = h2i.reshape(NPAD, F)

    xs2, asrc2, adst2 = _tc_layer(_tc_second, h, W2s, W2d, a2s, a2d)
    ex2, dparts2 = _sc_soft_call(asrc2.reshape(NPAD), adst2.reshape(NPAD),
                                 src, dst)
    out2 = _sc_agg_call(xs2.reshape(2 * NPAD, FH), src, dst, ex2, dparts2, b2)

    return out2.reshape(NPAD, F)[:N]
